# Initial kernel scaffold; baseline (speedup 1.0000x reference)
import jax
import jax.numpy as jnp
from jax.experimental import pallas as pl


def kernel(pre_x, x, edge_index, edge_type, num_prop, num_category, des_tensor, tweet_tensor, Wn, bn, Wc, bc, Wd, bd, Wt, bt, Wp, bp, Wi, bi, W1, root1, cb1, W2, root2, cb2, Wo1, bo1, Wo2, bo2):
    raise NotImplementedError("write your pallas kernel here")


# speedup vs baseline: 6.7417x; 6.7417x over previous
import functools

import jax
import jax.numpy as jnp
from jax import lax
from jax.experimental import pallas as pl
from jax.experimental.pallas import tpu as pltpu
from jax.experimental.pallas import tpu_sc as plsc

N = 10000
E = 320000
LM = 768
H = 160
HH = 80
NR = 2
S = NR * N
R2 = 2 * N

NC, NS = 2, 16
CH = 80
EPT = E // NS
NCHUNK = EPT // CH
RPT = S // NS
CPT = E // (NC * NS)

BN = 1000


def _leaky(x):
    return jnp.where(x >= 0, x, 0.01 * x)


def _sc_mesh():
    return plsc.VectorSubcoreMesh(
        core_axis_name="c", subcore_axis_name="s",
        num_cores=NC, num_subcores=NS)


def _sc_params():
    return pltpu.CompilerParams(use_tc_tiling_on_sc=False)


def _spmem_writeout(shared, hbm_dst, rbase, width):
    for r in range(RPT // 125):
        sl = pl.ds(rbase + r * 125, 125)
        pltpu.sync_copy(shared.at[sl], hbm_dst.at[sl])


def _sc_count_body(key, cnt_out, cnt, key_b, ones_b):
    c = lax.axis_index("c")
    s = lax.axis_index("s")

    zero16 = jnp.zeros((16,), jnp.float32)
    one16 = jnp.ones((16,), jnp.float32)

    def zinit(j, carry):
        ones_b[j, :] = zero16
        return carry
    lax.fori_loop(0, CH, zinit, 0)

    rbase = s * RPT
    for r in range(RPT // CH):
        pltpu.sync_copy(ones_b, cnt.at[pl.ds(rbase + r * CH, CH)])
    pltpu.sync_copy(ones_b.at[pl.ds(0, RPT - (RPT // CH) * CH)],
                    cnt.at[pl.ds(rbase + (RPT // CH) * CH,
                                 RPT - (RPT // CH) * CH)])

    def oinit(j, carry):
        ones_b[j, :] = one16
        return carry
    lax.fori_loop(0, CH, oinit, 0)

    plsc.subcore_barrier()

    ebase = c * (E // NC) + s * CPT

    def loop_body(ch, carry):
        pltpu.sync_copy(key.at[pl.ds(ebase + ch * CH, CH)], key_b)
        pltpu.sync_copy(ones_b, cnt.at[key_b], add=True)
        return carry
    lax.fori_loop(0, CPT // CH, loop_body, 0)

    plsc.subcore_barrier()
    _spmem_writeout(cnt, cnt_out.at[c], rbase, 16)


_sc_counts = pl.kernel(
    _sc_count_body,
    out_type=jax.ShapeDtypeStruct((NC, S, 16), jnp.float32),
    mesh=_sc_mesh(),
    scratch_types=[
        pltpu.VMEM_SHARED((S, 16), jnp.float32),
        pltpu.VMEM((CH,), jnp.int32),
        pltpu.VMEM((CH, 16), jnp.float32),
    ],
    compiler_params=_sc_params(),
)


def _sc_sum_body(h2, gidx, key, sums, acc,
                 idx_a, idx_b, key_a, key_b, row_a, row_b, sem_a, sem_b):
    c = lax.axis_index("c")
    s = lax.axis_index("s")

    zero16 = jnp.zeros((16,), jnp.float32)

    def zinit(j, carry):
        for k in range(HH // 16):
            row_a[j, pl.ds(k * 16, 16)] = zero16
        return carry
    lax.fori_loop(0, CH, zinit, 0)

    rbase = s * RPT
    for r in range(RPT // CH):
        pltpu.sync_copy(row_a, acc.at[pl.ds(rbase + r * CH, CH)])
    pltpu.sync_copy(row_a.at[pl.ds(0, RPT - (RPT // CH) * CH)],
                    acc.at[pl.ds(rbase + (RPT // CH) * CH,
                                 RPT - (RPT // CH) * CH)])

    plsc.subcore_barrier()

    ebase = s * EPT

    def load(ch, idxr, keyr):
        e0 = ebase + ch * CH
        pltpu.sync_copy(gidx.at[pl.ds(c * E + e0, CH)], idxr)
        pltpu.sync_copy(key.at[pl.ds(e0, CH)], keyr)

    def fire(idxr, rowr, sem):
        pltpu.async_copy(h2.at[idxr], rowr, sem)

    def wait(idxr, rowr, sem):
        pltpu.make_async_copy(h2.at[idxr], rowr, sem).wait()

    def process(keyr, rowr):
        pltpu.sync_copy(rowr, acc.at[keyr], add=True)

    load(0, idx_a, key_a)
    fire(idx_a, row_a, sem_a)
    load(1, idx_b, key_b)
    fire(idx_b, row_b, sem_b)

    def loop_body(jj, carry):
        ch0 = 2 * jj
        wait(idx_a, row_a, sem_a)
        process(key_a, row_a)
        load(ch0 + 2, idx_a, key_a)
        fire(idx_a, row_a, sem_a)
        wait(idx_b, row_b, sem_b)
        process(key_b, row_b)
        load(ch0 + 3, idx_b, key_b)
        fire(idx_b, row_b, sem_b)
        return carry
    lax.fori_loop(0, NCHUNK // 2 - 1, loop_body, 0)

    wait(idx_a, row_a, sem_a)
    process(key_a, row_a)
    wait(idx_b, row_b, sem_b)
    process(key_b, row_b)

    plsc.subcore_barrier()
    _spmem_writeout(acc, sums.at[c], rbase, HH)


_sc_pass = pl.kernel(
    _sc_sum_body,
    out_type=jax.ShapeDtypeStruct((NC, S, HH), jnp.float32),
    mesh=_sc_mesh(),
    scratch_types=[
        pltpu.VMEM_SHARED((S, HH), jnp.float32),
        pltpu.VMEM((CH,), jnp.int32),
        pltpu.VMEM((CH,), jnp.int32),
        pltpu.VMEM((CH,), jnp.int32),
        pltpu.VMEM((CH,), jnp.int32),
        pltpu.VMEM((CH, HH), jnp.float32),
        pltpu.VMEM((CH, HH), jnp.float32),
        pltpu.SemaphoreType.DMA,
        pltpu.SemaphoreType.DMA,
    ],
    compiler_params=_sc_params(),
)


def _dot(a, b):
    return jnp.dot(a, b, preferred_element_type=jnp.float32,
                   precision=lax.Precision.HIGHEST)


def _enc_body(np_, nc_, de_, tw_, px_, WnT, bn, WcT, bc, WdT, bd, WtT, bt,
              WpT, bp, WiT, bi, h_out):
    n = _leaky(_dot(np_[...], WnT[...]) + bn[...])
    c = _leaky(_dot(nc_[...], WcT[...]) + bc[...])
    d = _leaky(_dot(de_[...], WdT[...]) + bd[...])
    t = _leaky(_dot(tw_[...], WtT[...]) + bt[...])
    p = _leaky(_dot(px_[...], WpT[...]) + bp[...])
    h0 = jnp.concatenate([n, c, d, t, p], axis=1)
    h_out[...] = _leaky(_dot(h0, WiT[...]) + bi[...])


def _encoder(np_a, nc_a, de_a, tw_a, px_a, WnT, bn, WcT, bc, WdT, bd,
             WtT, bt, WpT, bp, WiT, bi):
    row = lambda i: (i, 0)
    full = lambda i: (0, 0)
    specs = [
        pl.BlockSpec((BN, 6), row), pl.BlockSpec((BN, 11), row),
        pl.BlockSpec((BN, LM), row), pl.BlockSpec((BN, LM), row),
        pl.BlockSpec((BN, LM), row),
        pl.BlockSpec((6, 32), full), pl.BlockSpec((1, 32), full),
        pl.BlockSpec((11, 32), full), pl.BlockSpec((1, 32), full),
        pl.BlockSpec((LM, 32), full), pl.BlockSpec((1, 32), full),
        pl.BlockSpec((LM, 32), full), pl.BlockSpec((1, 32), full),
        pl.BlockSpec((LM, 32), full), pl.BlockSpec((1, 32), full),
        pl.BlockSpec((H, H), full), pl.BlockSpec((1, H), full),
    ]
    return pl.pallas_call(
        _enc_body,
        grid=(N // BN,),
        in_specs=specs,
        out_specs=pl.BlockSpec((BN, H), row),
        out_shape=jax.ShapeDtypeStruct((N, H), jnp.float32),
    )(np_a, nc_a, de_a, tw_a, px_a, WnT, bn, WcT, bc, WdT, bd, WtT, bt,
      WpT, bp, WiT, bi)


def _mix(h, sv, cv, root, W0, W1, cb):
    s0 = jnp.concatenate([sv[0, 0], sv[1, 0]], axis=1)
    s1 = jnp.concatenate([sv[0, 1], sv[1, 1]], axis=1)
    c0 = jnp.max(cv[0, 0], axis=1, keepdims=True) + jnp.max(cv[1, 0], axis=1, keepdims=True)
    c1 = jnp.max(cv[0, 1], axis=1, keepdims=True) + jnp.max(cv[1, 1], axis=1, keepdims=True)
    inv0 = 1.0 / jnp.maximum(c0, 1.0)
    inv1 = 1.0 / jnp.maximum(c1, 1.0)
    return (_dot(h[...], root[...]) + cb[...]
            + _dot(s0, W0[...]) * inv0
            + _dot(s1, W1[...]) * inv1)


def _combine_body(h, sv, cv, root, W0, W1, cb, out):
    out[...] = _mix(h, sv, cv, root, W0, W1, cb)


def _combine_specs():
    row = lambda i: (i, 0)
    return [
        pl.BlockSpec((BN, H), row),
        pl.BlockSpec((2, 2, BN, HH), lambda i: (0, 0, i, 0)),
        pl.BlockSpec((2, 2, BN, 16), lambda i: (0, 0, i, 0)),
        pl.BlockSpec((H, H), lambda i: (0, 0)),
        pl.BlockSpec((H, H), lambda i: (0, 0)),
        pl.BlockSpec((H, H), lambda i: (0, 0)),
        pl.BlockSpec((1, H), lambda i: (0, 0)),
    ]


def _combine(h, sv, cv, root, W0, W1, cb):
    return pl.pallas_call(
        _combine_body,
        grid=(N // BN,),
        in_specs=_combine_specs(),
        out_specs=pl.BlockSpec((BN, H), lambda i: (i, 0)),
        out_shape=jax.ShapeDtypeStruct((N, H), jnp.float32),
    )(h, sv, cv, root, W0, W1, cb)


def _final_body(h, sv, cv, root, W0, W1, cb, Wo1T, bo1, Wo2T, bo2,
                logits, em):
    g = _mix(h, sv, cv, root, W0, W1, cb)
    e = _leaky(_dot(g, Wo1T[...]) + bo1[...])
    em[...] = e
    logits[...] = _dot(e, Wo2T[...]) + bo2[...]


def _final(h, sv, cv, root, W0, W1, cb, Wo1T, bo1, Wo2T, bo2):
    full = lambda i: (0, 0)
    specs = _combine_specs() + [
        pl.BlockSpec((H, 80), full), pl.BlockSpec((1, 80), full),
        pl.BlockSpec((80, 2), full), pl.BlockSpec((1, 2), full),
    ]
    return pl.pallas_call(
        _final_body,
        grid=(N // BN,),
        in_specs=specs,
        out_specs=[pl.BlockSpec((BN, 2), lambda i: (i, 0)),
                   pl.BlockSpec((BN, 80), lambda i: (i, 0))],
        out_shape=[jax.ShapeDtypeStruct((N, 2), jnp.float32),
                   jax.ShapeDtypeStruct((N, 80), jnp.float32)],
    )(h, sv, cv, root, W0, W1, cb, Wo1T, bo1, Wo2T, bo2)


def kernel(pre_x, x, edge_index, edge_type, num_prop, num_category,
           des_tensor, tweet_tensor, Wn, bn, Wc, bc, Wd, bd, Wt, bt, Wp, bp,
           Wi, bi, W1, root1, cb1, W2, root2, cb2, Wo1, bo1, Wo2, bo2):
    src = edge_index[0]
    dst = edge_index[1]
    gidx = jnp.concatenate([src * 2, src * 2 + 1])
    key = edge_type * N + dst

    h0 = _encoder(num_prop, num_category, des_tensor, tweet_tensor, pre_x,
                  Wn.T, bn.reshape(1, -1), Wc.T, bc.reshape(1, -1),
                  Wd.T, bd.reshape(1, -1), Wt.T, bt.reshape(1, -1),
                  Wp.T, bp.reshape(1, -1), Wi.T, bi.reshape(1, -1))

    cnt2 = _sc_counts(key)
    cv = cnt2.reshape(NC, NR, N, 16)
    sumsA = _sc_pass(h0.reshape(R2, HH), gidx, key)
    h1 = _combine(h0, sumsA.reshape(NC, NR, N, HH), cv,
                  root1, W1[0], W1[1], cb1.reshape(1, -1))

    sumsB = _sc_pass(h1.reshape(R2, HH), gidx, key)
    logits, em = _final(h1, sumsB.reshape(NC, NR, N, HH), cv,
                        root2, W2[0], W2[1], cb2.reshape(1, -1),
                        Wo1.T, bo1.reshape(1, -1), Wo2.T, bo2.reshape(1, -1))
    return (logits, em)
